# baseline (speedup 1.0000x reference)
import functools

import jax
import jax.numpy as jnp
from jax import lax
from jax.experimental import pallas as pl
from jax.experimental.pallas import tpu as pltpu
from jax.experimental.pallas import tpu_sc as plsc

_VOCAB = 1000000
_D = 64
_MAX_LEN = 200
_BATCH = 4096
_B = _BATCH * _MAX_LEN

_NC = 2
_NS = 16
_NW = _NC * _NS
_PER_W = _B // _NW
_C = 200
_NCHUNK = _PER_W // _C
_L = 16


def _positional() -> jax.Array:
    position = jnp.arange(0, _MAX_LEN, dtype=jnp.float32).reshape(-1, 1)
    div = jnp.exp(
        jnp.arange(0, _D, 2, dtype=jnp.float32) / _D
        * -jnp.log(jnp.float32(10000.0))
    )
    p = jnp.zeros((_MAX_LEN, _D), dtype=jnp.float32)
    p = p.at[:, 0::2].set(jnp.sin(position * div))
    p = p.at[:, 1::2].set(jnp.cos(position * div))
    return p


_mesh = plsc.VectorSubcoreMesh(core_axis_name="c", subcore_axis_name="s")


@functools.partial(
    pl.kernel,
    mesh=_mesh,
    out_type=jax.ShapeDtypeStruct((_B, _D), jnp.float32),
    scratch_types=[
        pltpu.SMEM((3, _C), jnp.int32),
        pltpu.VMEM_SHARED((_NS, 3, _C), jnp.int32),
        pltpu.VMEM((3, _C, _D), jnp.float32),
        pltpu.VMEM((_MAX_LEN, _D), jnp.float32),
        pltpu.SemaphoreType.DMA((3,)),
        pltpu.SemaphoreType.DMA((3,)),
        pltpu.SemaphoreType.DMA((3,)),
    ],
    compiler_params=pltpu.CompilerParams(use_tc_tiling_on_sc=False),
)
def _embed(x_hbm, table_hbm, p_hbm, out_hbm,
           idxs, idxv, rows, p_v, isem, gsem, ssem):
    sid = lax.axis_index("s")
    wid = sid * _NC + lax.axis_index("c")
    base = wid * _PER_W
    pltpu.sync_copy(p_hbm, p_v)

    def idx_load(k, slot):
        pltpu.async_copy(
            x_hbm.at[pl.ds(base + k * _C, _C)], idxv.at[sid].at[slot],
            isem.at[slot])

    def idx_wait(k, slot):
        pltpu.make_async_copy(
            x_hbm.at[pl.ds(base + k * _C, _C)], idxv.at[sid].at[slot],
            isem.at[slot]).wait()
        pltpu.sync_copy(idxv.at[sid].at[slot], idxs.at[slot])

    def fire_row(slot, rb, r):
        idx = idxs[slot, r]
        pltpu.async_copy(
            table_hbm.at[pl.ds(idx, 1)],
            rows.at[rb].at[pl.ds(r, 1)],
            gsem.at[rb],
        )

    def gather_drain(rb):
        pltpu.make_async_copy(
            table_hbm.at[pl.ds(0, _C)], rows.at[rb], gsem.at[rb]).wait()

    def store(k, rb):
        pltpu.async_copy(
            rows.at[rb], out_hbm.at[pl.ds(base + k * _C, _C)], ssem.at[rb])

    def store_wait(k, rb):
        pltpu.make_async_copy(
            rows.at[rb], out_hbm.at[pl.ds(base + k * _C, _C)],
            ssem.at[rb]).wait()

    def add_row(rb, r):
        for d in range(_D // _L):
            sl = pl.ds(d * _L, _L)
            rows[rb, r, sl] = rows[rb, r, sl] + p_v[r, sl]

    pltpu.sync_copy(x_hbm.at[pl.ds(base, _C)], idxv.at[sid].at[0])
    pltpu.sync_copy(idxv.at[sid].at[0], idxs.at[0])
    idx_load(1, 1)

    def fire_only(r, carry):
        fire_row(0, 0, r)
        return carry

    lax.fori_loop(0, _C, fire_only, 0)

    def chunk_body(k, carry):
        rb = lax.rem(k, 3)
        nb = lax.rem(k + 1, 3)
        fb = lax.rem(k + 2, 3)

        @pl.when(k + 1 < _NCHUNK)
        def _prep_next():
            idx_wait(k + 1, nb)

            @pl.when(k >= 2)
            def _drain_old_store():
                store_wait(k - 2, nb)

        gather_drain(rb)

        @pl.when(k + 1 < _NCHUNK)
        def _fused():
            def body(r, c2):
                fire_row(nb, nb, r)
                add_row(rb, r)
                return c2

            lax.fori_loop(0, _C, body, 0)

        @pl.when(k + 1 >= _NCHUNK)
        def _add_only():
            def body(r, c2):
                add_row(rb, r)
                return c2

            lax.fori_loop(0, _C, body, 0)

        store(k, rb)

        @pl.when(k + 2 < _NCHUNK)
        def _fire_next_idx():
            idx_load(k + 2, fb)

        return carry

    lax.fori_loop(0, _NCHUNK, chunk_body, 0)
    store_wait(_NCHUNK - 2, lax.rem(_NCHUNK - 2, 3))
    store_wait(_NCHUNK - 1, lax.rem(_NCHUNK - 1, 3))


def kernel(X, table):
    p = _positional()
    xf = X.reshape(-1)
    out = _embed(xf, table, p)
    return out.reshape(_BATCH, _MAX_LEN, _D)
